# Initial kernel scaffold; baseline (speedup 1.0000x reference)
import functools

import jax
import jax.numpy as jnp
from jax import lax
from jax.experimental import pallas as pl
from jax.experimental.pallas import tpu as pltpu
from jax.experimental.pallas import tpu_sc as plsc

VOCAB = 100000
EMBED = 32
BATCH = 16384
FIELDS = 26

N = BATCH * FIELDS
NW = 32
PER_W = N // NW
CHUNK = 1024
NCHUNK = PER_W // CHUNK
SUB = 128
NSUB = CHUNK // SUB
GROUPS = CHUNK // 16


def _body(table_hbm, idx_hbm, val_hbm, out_hbm, idx_v, val_v, rows_v, sem):
    cid = lax.axis_index("c")
    sid = lax.axis_index("s")
    wid = sid * 2 + cid
    base = wid * PER_W

    def chunk_step(c, _):
        r0 = base + c * CHUNK
        pltpu.sync_copy(idx_hbm.at[pl.ds(r0 // SUB, NSUB)], idx_v)
        pltpu.sync_copy(val_hbm.at[pl.ds(r0, CHUNK)], val_v)
        copies = []
        for j in range(NSUB):
            copies.append(
                pltpu.async_copy(
                    table_hbm.at[idx_v.at[j]],
                    rows_v.at[pl.ds(j * SUB, SUB)],
                    sem,
                )
            )
        for cp in copies:
            cp.wait()

        def mul_group(g, _):
            for j in range(16):
                r = g * 16 + j
                s = jnp.broadcast_to(val_v[r], (16,))
                rows_v[r, pl.ds(0, 16)] = rows_v[r, pl.ds(0, 16)] * s
                rows_v[r, pl.ds(16, 16)] = rows_v[r, pl.ds(16, 16)] * s
            return 0

        lax.fori_loop(0, GROUPS, mul_group, 0)

        pltpu.sync_copy(rows_v, out_hbm.at[pl.ds(r0, CHUNK)])
        return 0

    lax.fori_loop(0, NCHUNK, chunk_step, 0)


@jax.jit
def _run(table, idx2d, val):
    kern = functools.partial(
        pl.kernel,
        out_type=jax.ShapeDtypeStruct((N, EMBED), jnp.float32),
        mesh=plsc.VectorSubcoreMesh(core_axis_name="c", subcore_axis_name="s"),
        scratch_types=[
            pltpu.VMEM((NSUB, SUB), jnp.int32),
            pltpu.VMEM((CHUNK,), jnp.float32),
            pltpu.VMEM((CHUNK, EMBED), jnp.float32),
            pltpu.SemaphoreType.DMA,
        ],
    )(_body)
    return kern(table, idx2d, val)


def kernel(embed_index, embed_value, table):
    idx2d = embed_index.astype(jnp.int32).reshape(N // SUB, SUB)
    val = embed_value.reshape(N).astype(jnp.float32)
    out = _run(table, idx2d, val)
    return out.reshape(BATCH, FIELDS, EMBED)


# speedup vs baseline: 5.4707x; 5.4707x over previous
import functools

import jax
import jax.numpy as jnp
from jax import lax
from jax.experimental import pallas as pl
from jax.experimental.pallas import tpu as pltpu
from jax.experimental.pallas import tpu_sc as plsc

VOCAB = 100000
EMBED = 32
BATCH = 16384
FIELDS = 26

N = BATCH * FIELDS
NW = 32
PER_W = N // NW
CHUNK = 1024
NCHUNK = PER_W // CHUNK
SUB = 128
NSUB = CHUNK // SUB
GROUPS = CHUNK // 16


def _body(table_hbm, idx_hbm, val_hbm, out_hbm, idx_v, val_v, rows_v, sem):
    cid = lax.axis_index("c")
    sid = lax.axis_index("s")
    wid = sid * 2 + cid
    base = wid * PER_W

    def chunk_step(c, _):
        r0 = pl.multiple_of(base + c * CHUNK, CHUNK)
        pltpu.sync_copy(idx_hbm.at[pl.ds(pl.multiple_of(r0 // SUB, 8), NSUB)], idx_v)
        pltpu.sync_copy(val_hbm.at[pl.ds(r0, CHUNK)], val_v)
        copies = []
        for j in range(NSUB):
            copies.append(
                pltpu.async_copy(
                    table_hbm.at[idx_v.at[j]],
                    rows_v.at[pl.ds(j * SUB, SUB)],
                    sem,
                )
            )
        for cp in copies:
            cp.wait()

        def mul_group(g, _):
            v16 = val_v[pl.ds(g * 16, 16)]
            for j in range(16):
                r = g * 16 + j
                s = jnp.broadcast_to(v16[j], (16,))
                rows_v[r, pl.ds(0, 16)] = rows_v[r, pl.ds(0, 16)] * s
                rows_v[r, pl.ds(16, 16)] = rows_v[r, pl.ds(16, 16)] * s
            return 0

        lax.fori_loop(0, GROUPS, mul_group, 0)

        pltpu.sync_copy(rows_v, out_hbm.at[pl.ds(r0, CHUNK)])
        return 0

    lax.fori_loop(0, NCHUNK, chunk_step, 0)


@jax.jit
def _run(table, idx2d, val):
    kern = functools.partial(
        pl.kernel,
        out_type=jax.ShapeDtypeStruct((N, EMBED), jnp.float32),
        mesh=plsc.VectorSubcoreMesh(core_axis_name="c", subcore_axis_name="s"),
        scratch_types=[
            pltpu.VMEM((NSUB, SUB), jnp.int32),
            pltpu.VMEM((CHUNK,), jnp.float32),
            pltpu.VMEM((CHUNK, EMBED), jnp.float32),
            pltpu.SemaphoreType.DMA,
        ],
        compiler_params=pltpu.CompilerParams(use_tc_tiling_on_sc=False),
    )(_body)
    return kern(table, idx2d, val)


def kernel(embed_index, embed_value, table):
    idx2d = embed_index.astype(jnp.int32).reshape(N // SUB, SUB)
    val = embed_value.reshape(N).astype(jnp.float32)
    out = _run(table, idx2d, val)
    return out.reshape(BATCH, FIELDS, EMBED)
